# baseline (speedup 1.0000x reference)
import jax
import jax.numpy as jnp
import numpy as np
from jax.experimental import pallas as pl
from jax.experimental.pallas import tpu as pltpu

_NUM_TIME_MASKS = 10
_NUM_FREQ_MASKS = 2
_TIME_MASK_RATIO = 0.05
_MAX_FREQ_MASK_SIZE = 27

_BB = 8


def _mask_constants(frame: int, n_mels: int):
    rng = np.random.default_rng(0)
    f = int(rng.integers(0, _MAX_FREQ_MASK_SIZE + 1))
    f0 = rng.integers(0, n_mels - f, size=(_NUM_FREQ_MASKS,))
    fmask = np.ones((n_mels,), np.float32)
    if f > 0:
        for s in f0:
            fmask[s : s + f] = 0.0
    max_t = int(np.floor(_TIME_MASK_RATIO * frame))
    t = int(rng.integers(0, max_t + 1))
    t0 = rng.integers(0, frame - t, size=(_NUM_TIME_MASKS,))
    tmask = np.ones((frame,), np.float32)
    segs = []
    if t > 0:
        for s in sorted(int(v) for v in t0):
            tmask[s : s + t] = 0.0
            segs.append((s, s + t))
    runs, prev = [], 0
    for s, e in segs:
        if s > prev:
            runs.append((prev, s))
        prev = max(prev, e)
    if prev < frame:
        runs.append((prev, frame))
    plane = tmask[:, None] * fmask[None, :]
    return runs, segs, plane


def kernel(x):
    b, frame, n_mels = x.shape
    runs, segs, plane = _mask_constants(frame, n_mels)
    mask = jnp.asarray(plane)[None, :, :]
    nsteps = b // _BB
    nruns = len(runs)

    def body(x_hbm, m_ref, o_ref, buf, sems):
        i = pl.program_id(0)
        slot = jax.lax.rem(i, 5)

        def copy(step, slot, ridx):
            r0, r1 = runs[ridx]
            return pltpu.make_async_copy(
                x_hbm.at[pl.ds(step * _BB, _BB), pl.ds(r0, r1 - r0), :],
                buf.at[slot, :, pl.ds(r0, r1 - r0), :],
                sems.at[slot, ridx],
            )

        @pl.when(i == 0)
        def _():
            for s_ in range(4):
                for r in range(nruns):
                    copy(s_, s_, r).start()

        @pl.when(i + 4 < nsteps)
        def _():
            for r in range(nruns):
                copy(i + 4, jax.lax.rem(i + 4, 5), r).start()

        for m0, m1 in segs:
            o_ref[:, pl.ds(m0, m1 - m0), :] = jnp.zeros(
                (_BB, m1 - m0, n_mels), jnp.float32
            )
        for ridx, (r0, r1) in enumerate(runs):
            copy(i, slot, ridx).wait()
            o_ref[:, pl.ds(r0, r1 - r0), :] = jnp.where(
                m_ref[:, pl.ds(r0, r1 - r0), :] != 0.0,
                buf[slot, :, pl.ds(r0, r1 - r0), :],
                0.0,
            )

    return pl.pallas_call(
        body,
        grid=(nsteps,),
        in_specs=[
            pl.BlockSpec(memory_space=pl.ANY),
            pl.BlockSpec((1, frame, n_mels), lambda i: (0, 0, 0)),
        ],
        out_specs=pl.BlockSpec((_BB, frame, n_mels), lambda i: (i, 0, 0)),
        out_shape=jax.ShapeDtypeStruct(x.shape, x.dtype),
        scratch_shapes=[
            pltpu.VMEM((5, _BB, frame, n_mels), jnp.float32),
            pltpu.SemaphoreType.DMA((5, nruns)),
        ],
    )(x, mask)


# speedup vs baseline: 1.0031x; 1.0031x over previous
import jax
import jax.numpy as jnp
import numpy as np
from jax.experimental import pallas as pl
from jax.experimental.pallas import tpu as pltpu

_NUM_TIME_MASKS = 10
_NUM_FREQ_MASKS = 2
_TIME_MASK_RATIO = 0.05
_MAX_FREQ_MASK_SIZE = 27

_BB = 8


def _mask_constants(frame: int, n_mels: int):
    rng = np.random.default_rng(0)
    f = int(rng.integers(0, _MAX_FREQ_MASK_SIZE + 1))
    f0 = rng.integers(0, n_mels - f, size=(_NUM_FREQ_MASKS,))
    fmask = np.ones((n_mels,), np.float32)
    if f > 0:
        for s in f0:
            fmask[s : s + f] = 0.0
    max_t = int(np.floor(_TIME_MASK_RATIO * frame))
    t = int(rng.integers(0, max_t + 1))
    t0 = rng.integers(0, frame - t, size=(_NUM_TIME_MASKS,))
    tmask = np.ones((frame,), np.float32)
    segs = []
    if t > 0:
        for s in sorted(int(v) for v in t0):
            tmask[s : s + t] = 0.0
            segs.append((s, s + t))
    runs, prev = [], 0
    for s, e in segs:
        if s > prev:
            runs.append((prev, s))
        prev = max(prev, e)
    if prev < frame:
        runs.append((prev, frame))
    plane = tmask[:, None] * fmask[None, :]
    return runs, segs, plane


def kernel(x):
    b, frame, n_mels = x.shape
    runs, segs, plane = _mask_constants(frame, n_mels)
    mask = jnp.asarray(plane)[None, :, :]
    nsteps = b // _BB
    nruns = len(runs)

    def body(x_hbm, m_ref, o_ref, buf, sems):
        i = pl.program_id(0)
        slot = jax.lax.rem(i, 4)

        def copy(step, slot, ridx):
            r0, r1 = runs[ridx]
            return pltpu.make_async_copy(
                x_hbm.at[pl.ds(step * _BB, _BB), pl.ds(r0, r1 - r0), :],
                buf.at[slot, :, pl.ds(r0, r1 - r0), :],
                sems.at[slot, ridx],
            )

        @pl.when(i == 0)
        def _():
            for s_ in range(3):
                for r in range(nruns):
                    copy(s_, s_, r).start()

        @pl.when(i + 3 < nsteps)
        def _():
            for r in range(nruns):
                copy(i + 3, jax.lax.rem(i + 3, 4), r).start()

        for m0, m1 in segs:
            o_ref[:, pl.ds(m0, m1 - m0), :] = jnp.zeros(
                (_BB, m1 - m0, n_mels), jnp.float32
            )
        for ridx, (r0, r1) in enumerate(runs):
            copy(i, slot, ridx).wait()
            o_ref[:, pl.ds(r0, r1 - r0), :] = jnp.where(
                m_ref[:, pl.ds(r0, r1 - r0), :] != 0.0,
                buf[slot, :, pl.ds(r0, r1 - r0), :],
                0.0,
            )

    return pl.pallas_call(
        body,
        grid=(nsteps,),
        in_specs=[
            pl.BlockSpec(memory_space=pl.ANY),
            pl.BlockSpec((1, frame, n_mels), lambda i: (0, 0, 0)),
        ],
        out_specs=pl.BlockSpec((_BB, frame, n_mels), lambda i: (i, 0, 0)),
        out_shape=jax.ShapeDtypeStruct(x.shape, x.dtype),
        scratch_shapes=[
            pltpu.VMEM((4, _BB, frame, n_mels), jnp.float32),
            pltpu.SemaphoreType.DMA((4, nruns)),
        ],
    )(x, mask)
